# Initial kernel scaffold; baseline (speedup 1.0000x reference)
import jax
import jax.numpy as jnp
from jax.experimental import pallas as pl


def kernel(x, edge_index, W1, b1, W2, b2):
    raise NotImplementedError("write your pallas kernel here")


# speedup vs baseline: 3.0907x; 3.0907x over previous
import jax
import jax.numpy as jnp
from jax import lax
from jax.experimental import pallas as pl

BR = 400


def _tc1_body(deg_ref, x_ref, w1_ref, dis_ref, h1s_ref):
    deg = deg_ref[...] + 1.0
    dis = lax.rsqrt(deg)
    h1 = jnp.dot(x_ref[...], w1_ref[...], preferred_element_type=jnp.float32)
    dis_ref[...] = dis
    h1s_ref[...] = h1 * dis


def _tc2_body(agg_ref, h1s_ref, dis_ref, b1_ref, h2s_ref):
    pre = (agg_ref[...] + h1s_ref[...]) * dis_ref[...] + b1_ref[...]
    h2s_ref[...] = jnp.maximum(pre, 0.0) * dis_ref[...]


def _tc3_body(agg_ref, h2s_ref, dis_ref, w2_ref, b2_ref, out_ref):
    t = (agg_ref[...] + h2s_ref[...]) * dis_ref[...]
    o = jnp.dot(t, w2_ref[...], preferred_element_type=jnp.float32) + b2_ref[...]
    m = jnp.max(o, axis=1, keepdims=True)
    lse = jnp.log(jnp.sum(jnp.exp(o - m), axis=1, keepdims=True)) + m
    out_ref[...] = o - lse


def kernel(x, edge_index, W1, b1, W2, b2):
    N, F = x.shape
    H = W1.shape[1]
    C = W2.shape[1]

    src = edge_index[0].astype(jnp.int32)
    dst = edge_index[1].astype(jnp.int32)

    npad = -(-N // BR) * BR
    x_pad = jnp.zeros((npad, F), jnp.float32).at[:N].set(x.astype(jnp.float32))
    b1r = b1.astype(jnp.float32).reshape(1, H)
    b2r = b2.astype(jnp.float32).reshape(1, C)

    deg = jnp.zeros((N,), jnp.float32).at[dst].add(1.0)
    degb = jnp.zeros((npad, H), jnp.float32).at[:N].set(deg[:, None])

    grid = npad // BR
    dis, h1s = pl.pallas_call(
        _tc1_body,
        grid=(grid,),
        in_specs=[
            pl.BlockSpec((BR, H), lambda i: (i, 0)),
            pl.BlockSpec((BR, F), lambda i: (i, 0)),
            pl.BlockSpec((F, H), lambda i: (0, 0)),
        ],
        out_specs=[
            pl.BlockSpec((BR, H), lambda i: (i, 0)),
            pl.BlockSpec((BR, H), lambda i: (i, 0)),
        ],
        out_shape=[
            jax.ShapeDtypeStruct((npad, H), jnp.float32),
            jax.ShapeDtypeStruct((npad, H), jnp.float32),
        ],
    )(degb, x_pad, W1.astype(jnp.float32))

    acc1 = jnp.zeros((npad, H), jnp.float32).at[dst].add(h1s[src])

    h2s = pl.pallas_call(
        _tc2_body,
        grid=(grid,),
        in_specs=[
            pl.BlockSpec((BR, H), lambda i: (i, 0)),
            pl.BlockSpec((BR, H), lambda i: (i, 0)),
            pl.BlockSpec((BR, H), lambda i: (i, 0)),
            pl.BlockSpec((1, H), lambda i: (0, 0)),
        ],
        out_specs=pl.BlockSpec((BR, H), lambda i: (i, 0)),
        out_shape=jax.ShapeDtypeStruct((npad, H), jnp.float32),
    )(acc1, h1s, dis, b1r)

    acc2 = jnp.zeros((npad, H), jnp.float32).at[dst].add(h2s[src])

    out = pl.pallas_call(
        _tc3_body,
        grid=(N // BR,),
        in_specs=[
            pl.BlockSpec((BR, H), lambda i: (i, 0)),
            pl.BlockSpec((BR, H), lambda i: (i, 0)),
            pl.BlockSpec((BR, H), lambda i: (i, 0)),
            pl.BlockSpec((H, C), lambda i: (0, 0)),
            pl.BlockSpec((1, C), lambda i: (0, 0)),
        ],
        out_specs=pl.BlockSpec((BR, C), lambda i: (i, 0)),
        out_shape=jax.ShapeDtypeStruct((N, C), jnp.float32),
    )(acc2, h2s, dis, W2.astype(jnp.float32), b2r)

    return out


# speedup vs baseline: 3.1210x; 1.0098x over previous
import jax
import jax.numpy as jnp
from jax import lax
from jax.experimental import pallas as pl

BR = 2000


def _tc1_body(deg_ref, x_ref, w1_ref, dis_ref, h1s_ref):
    deg = deg_ref[...] + 1.0
    dis = lax.rsqrt(deg)
    h1 = jnp.dot(x_ref[...], w1_ref[...], preferred_element_type=jnp.float32)
    dis_ref[...] = dis
    h1s_ref[...] = h1 * dis


def _tc2_body(agg_ref, h1s_ref, dis_ref, b1_ref, h2s_ref):
    pre = (agg_ref[...] + h1s_ref[...]) * dis_ref[...] + b1_ref[...]
    h2s_ref[...] = jnp.maximum(pre, 0.0) * dis_ref[...]


def _tc3_body(agg_ref, h2s_ref, dis_ref, w2_ref, b2_ref, out_ref):
    t = (agg_ref[...] + h2s_ref[...]) * dis_ref[...]
    o = jnp.dot(t, w2_ref[...], preferred_element_type=jnp.float32) + b2_ref[...]
    m = jnp.max(o, axis=1, keepdims=True)
    lse = jnp.log(jnp.sum(jnp.exp(o - m), axis=1, keepdims=True)) + m
    out_ref[...] = o - lse


def kernel(x, edge_index, W1, b1, W2, b2):
    N, F = x.shape
    H = W1.shape[1]
    C = W2.shape[1]

    src = edge_index[0].astype(jnp.int32)
    dst = edge_index[1].astype(jnp.int32)

    npad = -(-N // BR) * BR
    xf = x.astype(jnp.float32)
    x_pad = xf if npad == N else jnp.zeros((npad, F), jnp.float32).at[:N].set(xf)
    b1r = b1.astype(jnp.float32).reshape(1, H)
    b2r = b2.astype(jnp.float32).reshape(1, C)

    deg = jnp.zeros((N,), jnp.float32).at[dst].add(1.0)
    degb = jnp.zeros((npad, H), jnp.float32).at[:N].set(deg[:, None])

    grid = npad // BR
    dis, h1s = pl.pallas_call(
        _tc1_body,
        grid=(grid,),
        in_specs=[
            pl.BlockSpec((BR, H), lambda i: (i, 0)),
            pl.BlockSpec((BR, F), lambda i: (i, 0)),
            pl.BlockSpec((F, H), lambda i: (0, 0)),
        ],
        out_specs=[
            pl.BlockSpec((BR, H), lambda i: (i, 0)),
            pl.BlockSpec((BR, H), lambda i: (i, 0)),
        ],
        out_shape=[
            jax.ShapeDtypeStruct((npad, H), jnp.float32),
            jax.ShapeDtypeStruct((npad, H), jnp.float32),
        ],
    )(degb, x_pad, W1.astype(jnp.float32))

    acc1 = jnp.zeros((npad, H), jnp.float32).at[dst].add(h1s[src])

    h2s = pl.pallas_call(
        _tc2_body,
        grid=(grid,),
        in_specs=[
            pl.BlockSpec((BR, H), lambda i: (i, 0)),
            pl.BlockSpec((BR, H), lambda i: (i, 0)),
            pl.BlockSpec((BR, H), lambda i: (i, 0)),
            pl.BlockSpec((1, H), lambda i: (0, 0)),
        ],
        out_specs=pl.BlockSpec((BR, H), lambda i: (i, 0)),
        out_shape=jax.ShapeDtypeStruct((npad, H), jnp.float32),
    )(acc1, h1s, dis, b1r)

    acc2 = jnp.zeros((npad, H), jnp.float32).at[dst].add(h2s[src])

    out = pl.pallas_call(
        _tc3_body,
        grid=(N // BR,),
        in_specs=[
            pl.BlockSpec((BR, H), lambda i: (i, 0)),
            pl.BlockSpec((BR, H), lambda i: (i, 0)),
            pl.BlockSpec((BR, H), lambda i: (i, 0)),
            pl.BlockSpec((H, C), lambda i: (0, 0)),
            pl.BlockSpec((1, C), lambda i: (0, 0)),
        ],
        out_specs=pl.BlockSpec((BR, C), lambda i: (i, 0)),
        out_shape=jax.ShapeDtypeStruct((N, C), jnp.float32),
    )(acc2, h2s, dis, W2.astype(jnp.float32), b2r)

    return out
